# Initial kernel scaffold; baseline (speedup 1.0000x reference)
import jax
import jax.numpy as jnp
from jax import lax
from jax.experimental import pallas as pl
from jax.experimental.pallas import tpu as pltpu
from jax.experimental.pallas import tpu_sc as plsc

N = 10000
E = 160000
F = 256
FH = 128
T = 4
NPAD = 10240
NC = 2
NS = 16
CH = 128
ECHUNKS = E // CH
KMAX = (ECHUNKS + NS - 1) // NS
RPT = NPAD // NS
RC = 160
EPT = E // NS

BN = 256
NB = NPAD // BN
BND = 400


def _mesh():
    return plsc.VectorSubcoreMesh(
        core_axis_name="c", subcore_axis_name="s", num_cores=NC)


def _deg_body(dst_hbm, deg_out, dstv, degv, red, outv, shared):
    c = lax.axis_index("c")
    s = lax.axis_index("s")

    @pl.when(c == 0)
    def _build():
        zeros = jnp.zeros((16,), jnp.float32)

        def zbody(i, carry):
            degv[pl.ds(i * 16, 16)] = zeros
            return carry

        lax.fori_loop(0, NPAD // 16, zbody, 0)
        pltpu.sync_copy(dst_hbm.at[pl.ds(s * EPT, EPT)], dstv)
        ones = jnp.ones((16,), jnp.float32)

        def hbody(i, carry):
            idx = dstv[pl.ds(i * 16, 16)]
            plsc.addupdate_scatter(degv, [idx], ones)
            return carry

        lax.fori_loop(0, EPT // 16, hbody, 0)
        pltpu.sync_copy(degv, shared.at[s])

    plsc.subcore_barrier()

    @pl.when(c == 0)
    def _reduce():
        base = s * RPT
        for k in range(NS):
            pltpu.sync_copy(shared.at[k, pl.ds(base, RPT)], red.at[k])

        def rbody(i, carry):
            acc = red[0, pl.ds(i * 16, 16)]
            for k in range(1, NS):
                acc = acc + red[k, pl.ds(i * 16, 16)]
            outv[pl.ds(i * 16, 16)] = acc
            return carry

        lax.fori_loop(0, RPT // 16, rbody, 0)
        pltpu.sync_copy(outv, deg_out.at[pl.ds(base, RPT)])


def _sc_degree(dst):
    return pl.kernel(
        _deg_body,
        out_type=jax.ShapeDtypeStruct((NPAD,), jnp.float32),
        mesh=_mesh(),
        scratch_types=[
            pltpu.VMEM((EPT,), jnp.int32),
            pltpu.VMEM((NPAD,), jnp.float32),
            pltpu.VMEM((NS, RPT), jnp.float32),
            pltpu.VMEM((RPT,), jnp.float32),
            pltpu.VMEM_SHARED((NS, NPAD), jnp.float32),
        ],
    )(dst)


def _agg_body(hws_hbm, src_hbm, dst_hbm, out_hbm,
              srcv, dstv, idxv, rows, bounce, shared, sem):
    c = lax.axis_index("c")
    s = lax.axis_index("s")
    rbase = s * RPT

    for t in range(T):
        tb = (c * T + t) * NPAD
        for j in range(RPT // RC):
            pltpu.sync_copy(hws_hbm.at[pl.ds(tb + rbase + j * RC, RC)], bounce)
            pltpu.sync_copy(bounce, shared.at[pl.ds(rbase + j * RC, RC)])
        plsc.subcore_barrier()

        def ebody(k, carry):
            chunk = k * NS + s

            @pl.when(chunk < ECHUNKS)
            def _():
                off = pl.multiple_of(chunk * CH, CH)
                pltpu.sync_copy(src_hbm.at[pl.ds(off, CH)], srcv)
                pltpu.sync_copy(dst_hbm.at[pl.ds(off, CH)], dstv)
                for j in range(CH // 16):
                    idxv[pl.ds(j * 16, 16)] = srcv[pl.ds(j * 16, 16)] + tb
                pltpu.async_copy(hws_hbm.at[idxv], rows, sem).wait()
                pltpu.sync_copy(rows, shared.at[dstv], add=True)

            return carry

        lax.fori_loop(0, KMAX, ebody, 0)
        plsc.subcore_barrier()

        for j in range(RPT // RC):
            pltpu.sync_copy(shared.at[pl.ds(rbase + j * RC, RC)], bounce)
            pltpu.sync_copy(bounce, out_hbm.at[c, t, pl.ds(rbase + j * RC, RC)])


def _sc_aggregate(hws_flat, src, dst):
    return pl.kernel(
        _agg_body,
        out_type=jax.ShapeDtypeStruct((NC, T, NPAD, FH), jnp.float32),
        mesh=_mesh(),
        scratch_types=[
            pltpu.VMEM((CH,), jnp.int32),
            pltpu.VMEM((CH,), jnp.int32),
            pltpu.VMEM((CH,), jnp.int32),
            pltpu.VMEM((CH, FH), jnp.float32),
            pltpu.VMEM((RC, FH), jnp.float32),
            pltpu.VMEM_SHARED((NPAD, FH), jnp.float32),
            pltpu.SemaphoreType.DMA,
        ],
    )(hws_flat, src, dst)


def _dinv_body(deg_ref, o_ref):
    o_ref[...] = lax.rsqrt(deg_ref[...] + 1.0)


def _tc_dinv(deg):
    out = pl.pallas_call(
        _dinv_body,
        out_shape=jax.ShapeDtypeStruct((NPAD // 128, 128), jnp.float32),
    )(deg.reshape(NPAD // 128, 128))
    return out.reshape(NPAD, 1)


def _mm1_body(h_ref, w_ref, dinv_ref, o_ref):
    hw = lax.dot_general(h_ref[0], w_ref[...], (((1,), (1,)), ((), ())),
                         preferred_element_type=jnp.float32)
    hw = hw * dinv_ref[...]
    o_ref[0, 0] = hw[:, :FH]
    o_ref[1, 0] = hw[:, FH:]


def _tc_mm1(h, W, dinv):
    return pl.pallas_call(
        _mm1_body,
        grid=(T, NB),
        in_specs=[
            pl.BlockSpec((1, BN, F), lambda t, nb: (t, nb, 0)),
            pl.BlockSpec((F, F), lambda t, nb: (0, 0)),
            pl.BlockSpec((BN, 1), lambda t, nb: (nb, 0)),
        ],
        out_specs=pl.BlockSpec((NC, 1, BN, FH), lambda t, nb: (0, t, nb, 0)),
        out_shape=jax.ShapeDtypeStruct((NC, T, NPAD, FH), jnp.float32),
    )(h, W, dinv)


def _mm2_body(accl_ref, accr_ref, w_ref, dinv_ref, b_ref, o_ref):
    acc = jnp.concatenate([accl_ref[0, 0], accr_ref[0, 0]], axis=1)
    h = jnp.maximum(acc * dinv_ref[...] + b_ref[...], 0.0)
    hw = lax.dot_general(h, w_ref[...], (((1,), (1,)), ((), ())),
                         preferred_element_type=jnp.float32)
    hw = hw * dinv_ref[...]
    o_ref[0, 0] = hw[:, :FH]
    o_ref[1, 0] = hw[:, FH:]


def _tc_mm2(acc, W, dinv, b):
    return pl.pallas_call(
        _mm2_body,
        grid=(T, NB),
        in_specs=[
            pl.BlockSpec((1, 1, BN, FH), lambda t, nb: (0, t, nb, 0)),
            pl.BlockSpec((1, 1, BN, FH), lambda t, nb: (1, t, nb, 0)),
            pl.BlockSpec((F, F), lambda t, nb: (0, 0)),
            pl.BlockSpec((BN, 1), lambda t, nb: (nb, 0)),
            pl.BlockSpec((1, F), lambda t, nb: (0, 0)),
        ],
        out_specs=pl.BlockSpec((NC, 1, BN, FH), lambda t, nb: (0, t, nb, 0)),
        out_shape=jax.ShapeDtypeStruct((NC, T, NPAD, FH), jnp.float32),
    )(acc, acc, W, dinv, b.reshape(1, F))


def _fin_body(accl_ref, accr_ref, dinv_ref, b_ref, o_ref):
    acc = jnp.concatenate([accl_ref[0, 0], accr_ref[0, 0]], axis=1)
    out = jnp.maximum(acc * dinv_ref[...] + b_ref[...], 0.0)
    o_ref[...] = out.reshape(BND, 1, F)


def _tc_finalize(acc, dinv, b):
    return pl.pallas_call(
        _fin_body,
        grid=(N // BND, T),
        in_specs=[
            pl.BlockSpec((1, 1, BND, FH), lambda nb, t: (0, t, nb, 0)),
            pl.BlockSpec((1, 1, BND, FH), lambda nb, t: (1, t, nb, 0)),
            pl.BlockSpec((BND, 1), lambda nb, t: (nb, 0)),
            pl.BlockSpec((1, F), lambda nb, t: (0, 0)),
        ],
        out_specs=pl.BlockSpec((BND, 1, F), lambda nb, t: (nb, t, 0)),
        out_shape=jax.ShapeDtypeStruct((N, T, F), jnp.float32),
    )(acc, acc, dinv, b.reshape(1, F))


def kernel(x, edge_index, W1, b1, W2, b2):
    src = edge_index[0]
    dst = edge_index[1]
    h1 = jnp.pad(jnp.transpose(x, (2, 0, 1)), ((0, 0), (0, NPAD - N), (0, 0)))

    deg = _sc_degree(dst)
    dinv = _tc_dinv(deg)

    hws1 = _tc_mm1(h1, W1, dinv)
    acc1 = _sc_aggregate(hws1.reshape(NC * T * NPAD, FH), src, dst)
    hws2 = _tc_mm2(acc1, W2, dinv, b1)
    acc2 = _sc_aggregate(hws2.reshape(NC * T * NPAD, FH), src, dst)
    return _tc_finalize(acc2, dinv, b2)


# speedup vs baseline: 18.6053x; 18.6053x over previous
import jax
import jax.numpy as jnp
from jax import lax
from jax.experimental import pallas as pl
from jax.experimental.pallas import tpu as pltpu
from jax.experimental.pallas import tpu_sc as plsc

N = 10000
E = 160000
F = 256
FH = 128
T = 4
NPAD = 10240
NC = 2
NS = 16
CH = 128
ECHUNKS = E // CH
KMAX = (ECHUNKS + NS - 1) // NS
RPT = NPAD // NS
RC = 160
EPT = E // NS

BN = 256
NB = NPAD // BN
BND = 400


def _mesh():
    return plsc.VectorSubcoreMesh(
        core_axis_name="c", subcore_axis_name="s", num_cores=NC)


_SC_PARAMS = pltpu.CompilerParams(needs_layout_passes=False)


def _deg_body(dst_hbm, deg_out, dstv, degv, red, outv, shared):
    c = lax.axis_index("c")
    s = lax.axis_index("s")

    @pl.when(c == 0)
    def _build():
        zeros = jnp.zeros((16,), jnp.float32)

        def zbody(i, carry):
            degv[pl.ds(i * 16, 16)] = zeros
            return carry

        lax.fori_loop(0, NPAD // 16, zbody, 0)
        pltpu.sync_copy(dst_hbm.at[pl.ds(s * EPT, EPT)], dstv)
        ones = jnp.ones((16,), jnp.float32)

        def hbody(i, carry):
            idx = dstv[pl.ds(i * 16, 16)]
            plsc.addupdate_scatter(degv, [idx], ones)
            return carry

        lax.fori_loop(0, EPT // 16, hbody, 0)
        pltpu.sync_copy(degv, shared.at[s])

    plsc.subcore_barrier()

    @pl.when(c == 0)
    def _reduce():
        base = s * RPT
        for k in range(NS):
            pltpu.sync_copy(shared.at[k, pl.ds(base, RPT)], red.at[k])

        def rbody(i, carry):
            acc = red[0, pl.ds(i * 16, 16)]
            for k in range(1, NS):
                acc = acc + red[k, pl.ds(i * 16, 16)]
            outv[pl.ds(i * 16, 16)] = acc
            return carry

        lax.fori_loop(0, RPT // 16, rbody, 0)
        pltpu.sync_copy(outv, deg_out.at[pl.ds(base, RPT)])


def _sc_degree(dst):
    return pl.kernel(
        _deg_body,
        out_type=jax.ShapeDtypeStruct((NPAD,), jnp.float32),
        mesh=_mesh(),
        scratch_types=[
            pltpu.VMEM((EPT,), jnp.int32),
            pltpu.VMEM((NPAD,), jnp.float32),
            pltpu.VMEM((NS, RPT), jnp.float32),
            pltpu.VMEM((RPT,), jnp.float32),
            pltpu.VMEM_SHARED((NS, NPAD), jnp.float32),
        ],
        compiler_params=_SC_PARAMS,
    )(dst)


def _agg_body(hws_hbm, src_hbm, dst_hbm, out_hbm,
              srcv, dstv, idxv, rows, bounce, shared, sem):
    c = lax.axis_index("c")
    s = lax.axis_index("s")
    rbase = s * RPT

    for t in range(T):
        tb = (c * T + t) * NPAD
        for j in range(RPT // RC):
            pltpu.sync_copy(hws_hbm.at[pl.ds(tb + rbase + j * RC, RC)], bounce)
            pltpu.sync_copy(bounce, shared.at[pl.ds(rbase + j * RC, RC)])
        plsc.subcore_barrier()

        def ebody(k, carry):
            chunk = k * NS + s

            @pl.when(chunk < ECHUNKS)
            def _():
                off = pl.multiple_of(chunk * CH, CH)
                pltpu.sync_copy(src_hbm.at[pl.ds(off, CH)], srcv)
                pltpu.sync_copy(dst_hbm.at[pl.ds(off, CH)], dstv)
                for j in range(CH // 16):
                    idxv[pl.ds(j * 16, 16)] = srcv[pl.ds(j * 16, 16)] + tb
                pltpu.async_copy(hws_hbm.at[idxv], rows, sem).wait()
                pltpu.sync_copy(rows, shared.at[dstv], add=True)

            return carry

        lax.fori_loop(0, KMAX, ebody, 0)
        plsc.subcore_barrier()

        for j in range(RPT // RC):
            pltpu.sync_copy(shared.at[pl.ds(rbase + j * RC, RC)], bounce)
            pltpu.sync_copy(bounce, out_hbm.at[c, t, pl.ds(rbase + j * RC, RC)])


def _sc_aggregate(hws_flat, src, dst):
    return pl.kernel(
        _agg_body,
        out_type=jax.ShapeDtypeStruct((NC, T, NPAD, FH), jnp.float32),
        mesh=_mesh(),
        scratch_types=[
            pltpu.VMEM((CH,), jnp.int32),
            pltpu.VMEM((CH,), jnp.int32),
            pltpu.VMEM((CH,), jnp.int32),
            pltpu.VMEM((CH, FH), jnp.float32),
            pltpu.VMEM((RC, FH), jnp.float32),
            pltpu.VMEM_SHARED((NPAD, FH), jnp.float32),
            pltpu.SemaphoreType.DMA,
        ],
        compiler_params=_SC_PARAMS,
    )(hws_flat, src, dst)


def _dinv_body(deg_ref, o_ref):
    o_ref[...] = lax.rsqrt(deg_ref[...] + 1.0)


def _tc_dinv(deg):
    out = pl.pallas_call(
        _dinv_body,
        out_shape=jax.ShapeDtypeStruct((NPAD // 128, 128), jnp.float32),
    )(deg.reshape(NPAD // 128, 128))
    return out.reshape(NPAD, 1)


def _mm1_body(h_ref, w_ref, dinv_ref, o_ref):
    hw = lax.dot_general(h_ref[0], w_ref[...], (((1,), (1,)), ((), ())),
                         preferred_element_type=jnp.float32)
    hw = hw * dinv_ref[...]
    o_ref[0, 0] = hw[:, :FH]
    o_ref[1, 0] = hw[:, FH:]


def _tc_mm1(h, W, dinv):
    return pl.pallas_call(
        _mm1_body,
        grid=(T, NB),
        in_specs=[
            pl.BlockSpec((1, BN, F), lambda t, nb: (t, nb, 0)),
            pl.BlockSpec((F, F), lambda t, nb: (0, 0)),
            pl.BlockSpec((BN, 1), lambda t, nb: (nb, 0)),
        ],
        out_specs=pl.BlockSpec((NC, 1, BN, FH), lambda t, nb: (0, t, nb, 0)),
        out_shape=jax.ShapeDtypeStruct((NC, T, NPAD, FH), jnp.float32),
    )(h, W, dinv)


def _mm2_body(accl_ref, accr_ref, w_ref, dinv_ref, b_ref, o_ref):
    acc = jnp.concatenate([accl_ref[0, 0], accr_ref[0, 0]], axis=1)
    h = jnp.maximum(acc * dinv_ref[...] + b_ref[...], 0.0)
    hw = lax.dot_general(h, w_ref[...], (((1,), (1,)), ((), ())),
                         preferred_element_type=jnp.float32)
    hw = hw * dinv_ref[...]
    o_ref[0, 0] = hw[:, :FH]
    o_ref[1, 0] = hw[:, FH:]


def _tc_mm2(acc, W, dinv, b):
    return pl.pallas_call(
        _mm2_body,
        grid=(T, NB),
        in_specs=[
            pl.BlockSpec((1, 1, BN, FH), lambda t, nb: (0, t, nb, 0)),
            pl.BlockSpec((1, 1, BN, FH), lambda t, nb: (1, t, nb, 0)),
            pl.BlockSpec((F, F), lambda t, nb: (0, 0)),
            pl.BlockSpec((BN, 1), lambda t, nb: (nb, 0)),
            pl.BlockSpec((1, F), lambda t, nb: (0, 0)),
        ],
        out_specs=pl.BlockSpec((NC, 1, BN, FH), lambda t, nb: (0, t, nb, 0)),
        out_shape=jax.ShapeDtypeStruct((NC, T, NPAD, FH), jnp.float32),
    )(acc, acc, W, dinv, b.reshape(1, F))


def _fin_body(accl_ref, accr_ref, dinv_ref, b_ref, o_ref):
    for t in range(T):
        acc = jnp.concatenate([accl_ref[0, t], accr_ref[0, t]], axis=1)
        o_ref[:, t, :] = jnp.maximum(acc * dinv_ref[...] + b_ref[...], 0.0)


def _tc_finalize(acc, dinv, b):
    return pl.pallas_call(
        _fin_body,
        grid=(N // BND,),
        in_specs=[
            pl.BlockSpec((1, T, BND, FH), lambda nb: (0, 0, nb, 0)),
            pl.BlockSpec((1, T, BND, FH), lambda nb: (1, 0, nb, 0)),
            pl.BlockSpec((BND, 1), lambda nb: (nb, 0)),
            pl.BlockSpec((1, F), lambda nb: (0, 0)),
        ],
        out_specs=pl.BlockSpec((BND, T, F), lambda nb: (nb, 0, 0)),
        out_shape=jax.ShapeDtypeStruct((N, T, F), jnp.float32),
    )(acc, acc, dinv, b.reshape(1, F))


def kernel(x, edge_index, W1, b1, W2, b2):
    src = edge_index[0]
    dst = edge_index[1]
    h1 = jnp.pad(jnp.transpose(x, (2, 0, 1)), ((0, 0), (0, NPAD - N), (0, 0)))

    deg = _sc_degree(dst)
    dinv = _tc_dinv(deg)

    hws1 = _tc_mm1(h1, W1, dinv)
    acc1 = _sc_aggregate(hws1.reshape(NC * T * NPAD, FH), src, dst)
    hws2 = _tc_mm2(acc1, W2, dinv, b1)
    acc2 = _sc_aggregate(hws2.reshape(NC * T * NPAD, FH), src, dst)
    return _tc_finalize(acc2, dinv, b2)


# speedup vs baseline: 33.8227x; 1.8179x over previous
import jax
import jax.numpy as jnp
from jax import lax
from jax.experimental import pallas as pl
from jax.experimental.pallas import tpu as pltpu
from jax.experimental.pallas import tpu_sc as plsc

N = 10000
E = 160000
F = 256
FH = 128
T = 4
NPAD = 10240
NC = 2
NS = 16
CH = 128
ECHUNKS = E // CH
KMAX = (ECHUNKS + NS - 1) // NS
RPT = NPAD // NS
RC = 160
EPT = E // NS

BN = 256
NB = NPAD // BN
BND = 400


def _mesh():
    return plsc.VectorSubcoreMesh(
        core_axis_name="c", subcore_axis_name="s", num_cores=NC)


_SC_PARAMS = pltpu.CompilerParams(needs_layout_passes=False)


def _deg_body(dst_hbm, deg_out, dstv, degv, red, outv, shared):
    c = lax.axis_index("c")
    s = lax.axis_index("s")

    @pl.when(c == 0)
    def _build():
        zeros = jnp.zeros((16,), jnp.float32)

        def zbody(i, carry):
            degv[pl.ds(i * 16, 16)] = zeros
            return carry

        lax.fori_loop(0, NPAD // 16, zbody, 0)
        pltpu.sync_copy(dst_hbm.at[pl.ds(s * EPT, EPT)], dstv)
        ones = jnp.ones((16,), jnp.float32)

        def hbody(i, carry):
            idx = dstv[pl.ds(i * 16, 16)]
            plsc.addupdate_scatter(degv, [idx], ones)
            return carry

        lax.fori_loop(0, EPT // 16, hbody, 0)
        pltpu.sync_copy(degv, shared.at[s])

    plsc.subcore_barrier()

    @pl.when(c == 0)
    def _reduce():
        base = s * RPT
        for k in range(NS):
            pltpu.sync_copy(shared.at[k, pl.ds(base, RPT)], red.at[k])

        def rbody(i, carry):
            acc = red[0, pl.ds(i * 16, 16)]
            for k in range(1, NS):
                acc = acc + red[k, pl.ds(i * 16, 16)]
            outv[pl.ds(i * 16, 16)] = acc
            return carry

        lax.fori_loop(0, RPT // 16, rbody, 0)
        pltpu.sync_copy(outv, deg_out.at[pl.ds(base, RPT)])


def _sc_degree(dst):
    return pl.kernel(
        _deg_body,
        out_type=jax.ShapeDtypeStruct((NPAD,), jnp.float32),
        mesh=_mesh(),
        scratch_types=[
            pltpu.VMEM((EPT,), jnp.int32),
            pltpu.VMEM((NPAD,), jnp.float32),
            pltpu.VMEM((NS, RPT), jnp.float32),
            pltpu.VMEM((RPT,), jnp.float32),
            pltpu.VMEM_SHARED((NS, NPAD), jnp.float32),
        ],
        compiler_params=_SC_PARAMS,
    )(dst)


CPT = EPT // CH
REM = EPT - CPT * CH
SUP = 6
NSUP = CPT // SUP
EBLK = SUP * CH


def _agg_body(hws_hbm, src_hbm, dst_hbm, out_hbm,
              srcblk, dstblk, idxg, idxs, rows,
              idxgr, idxsr, rowsr, shared, gsem, ssem):
    c = lax.axis_index("c")
    s = lax.axis_index("s")
    rbase = s * RPT
    ebase = s * EPT

    def _gather(r):
        return pltpu.make_async_copy(
            hws_hbm.at[idxg.at[r]], rows.at[r], gsem)

    def _scatter(r, q):
        return pltpu.make_async_copy(
            rows.at[r], shared.at[idxs.at[q]], ssem)

    for t in range(T):
        tb = (c * T + t) * NPAD
        for j in range(RPT // CH):
            r = j % 2
            pltpu.async_copy(
                hws_hbm.at[pl.ds(tb + rbase + j * CH, CH)],
                rows.at[r], gsem).wait()
            pltpu.sync_copy(rows.at[r], shared.at[pl.ds(rbase + j * CH, CH)])
        plsc.subcore_barrier()

        def sbody(sc, carry):
            eoff = ebase + sc * EBLK
            pltpu.sync_copy(src_hbm.at[pl.ds(eoff, EBLK)], srcblk)
            pltpu.sync_copy(dst_hbm.at[pl.ds(eoff, EBLK)], dstblk)
            for j in range(SUP):
                k = sc * SUP + j
                r = j % 2

                @pl.when(k >= 2)
                def _():
                    _scatter(r, (j + 4) % SUP).wait()

                for i in range(CH // 16):
                    idxg[r, pl.ds(i * 16, 16)] = (
                        srcblk[pl.ds(j * CH + i * 16, 16)] + tb)
                    idxs[j, pl.ds(i * 16, 16)] = (
                        dstblk[pl.ds(j * CH + i * 16, 16)])
                _gather(r).start()

                @pl.when(k >= 1)
                def _():
                    _gather(1 - r).wait()
                    _scatter(1 - r, (j + 5) % SUP).start(add=True)

            return carry

        lax.fori_loop(0, NSUP, sbody, 0)
        _gather(1).wait()
        _scatter(1, SUP - 1).start(add=True)
        _scatter(0, SUP - 2).wait()
        _scatter(1, SUP - 1).wait()

        pltpu.sync_copy(src_hbm.at[pl.ds(ebase + CPT * CH, REM)], idxgr)
        pltpu.sync_copy(dst_hbm.at[pl.ds(ebase + CPT * CH, REM)], idxsr)
        idxgr[...] = idxgr[...] + tb
        pltpu.async_copy(hws_hbm.at[idxgr], rowsr, gsem).wait()
        pltpu.sync_copy(rowsr, shared.at[idxsr], add=True)
        plsc.subcore_barrier()

        for j in range(RPT // CH):
            r = j % 2
            pltpu.sync_copy(shared.at[pl.ds(rbase + j * CH, CH)], rows.at[r])
            pltpu.async_copy(
                rows.at[r],
                out_hbm.at[c, t, pl.ds(rbase + j * CH, CH)], gsem).wait()


def _sc_aggregate(hws_flat, src, dst):
    return pl.kernel(
        _agg_body,
        out_type=jax.ShapeDtypeStruct((NC, T, NPAD, FH), jnp.float32),
        mesh=_mesh(),
        scratch_types=[
            pltpu.VMEM((EBLK,), jnp.int32),
            pltpu.VMEM((EBLK,), jnp.int32),
            pltpu.VMEM((2, CH), jnp.int32),
            pltpu.VMEM((SUP, CH), jnp.int32),
            pltpu.VMEM((2, CH, FH), jnp.float32),
            pltpu.VMEM((REM,), jnp.int32),
            pltpu.VMEM((REM,), jnp.int32),
            pltpu.VMEM((REM, FH), jnp.float32),
            pltpu.VMEM_SHARED((NPAD, FH), jnp.float32),
            pltpu.SemaphoreType.DMA,
            pltpu.SemaphoreType.DMA,
        ],
        compiler_params=_SC_PARAMS,
    )(hws_flat, src, dst)


def _dinv_body(deg_ref, o_ref):
    o_ref[...] = lax.rsqrt(deg_ref[...] + 1.0)


def _tc_dinv(deg):
    out = pl.pallas_call(
        _dinv_body,
        out_shape=jax.ShapeDtypeStruct((NPAD // 128, 128), jnp.float32),
    )(deg.reshape(NPAD // 128, 128))
    return out.reshape(NPAD, 1)


def _mm1_body(h_ref, w_ref, dinv_ref, o_ref):
    hw = lax.dot_general(h_ref[0], w_ref[...], (((1,), (1,)), ((), ())),
                         preferred_element_type=jnp.float32)
    hw = hw * dinv_ref[...]
    o_ref[0, 0] = hw[:, :FH]
    o_ref[1, 0] = hw[:, FH:]


def _tc_mm1(h, W, dinv):
    return pl.pallas_call(
        _mm1_body,
        grid=(T, NB),
        in_specs=[
            pl.BlockSpec((1, BN, F), lambda t, nb: (t, nb, 0)),
            pl.BlockSpec((F, F), lambda t, nb: (0, 0)),
            pl.BlockSpec((BN, 1), lambda t, nb: (nb, 0)),
        ],
        out_specs=pl.BlockSpec((NC, 1, BN, FH), lambda t, nb: (0, t, nb, 0)),
        out_shape=jax.ShapeDtypeStruct((NC, T, NPAD, FH), jnp.float32),
    )(h, W, dinv)


def _mm2_body(accl_ref, accr_ref, w_ref, dinv_ref, b_ref, o_ref):
    acc = jnp.concatenate([accl_ref[0, 0], accr_ref[0, 0]], axis=1)
    h = jnp.maximum(acc * dinv_ref[...] + b_ref[...], 0.0)
    hw = lax.dot_general(h, w_ref[...], (((1,), (1,)), ((), ())),
                         preferred_element_type=jnp.float32)
    hw = hw * dinv_ref[...]
    o_ref[0, 0] = hw[:, :FH]
    o_ref[1, 0] = hw[:, FH:]


def _tc_mm2(acc, W, dinv, b):
    return pl.pallas_call(
        _mm2_body,
        grid=(T, NB),
        in_specs=[
            pl.BlockSpec((1, 1, BN, FH), lambda t, nb: (0, t, nb, 0)),
            pl.BlockSpec((1, 1, BN, FH), lambda t, nb: (1, t, nb, 0)),
            pl.BlockSpec((F, F), lambda t, nb: (0, 0)),
            pl.BlockSpec((BN, 1), lambda t, nb: (nb, 0)),
            pl.BlockSpec((1, F), lambda t, nb: (0, 0)),
        ],
        out_specs=pl.BlockSpec((NC, 1, BN, FH), lambda t, nb: (0, t, nb, 0)),
        out_shape=jax.ShapeDtypeStruct((NC, T, NPAD, FH), jnp.float32),
    )(acc, acc, W, dinv, b.reshape(1, F))


def _fin_body(accl_ref, accr_ref, dinv_ref, b_ref, o_ref):
    for t in range(T):
        acc = jnp.concatenate([accl_ref[0, t], accr_ref[0, t]], axis=1)
        o_ref[:, t, :] = jnp.maximum(acc * dinv_ref[...] + b_ref[...], 0.0)


def _tc_finalize(acc, dinv, b):
    return pl.pallas_call(
        _fin_body,
        grid=(N // BND,),
        in_specs=[
            pl.BlockSpec((1, T, BND, FH), lambda nb: (0, 0, nb, 0)),
            pl.BlockSpec((1, T, BND, FH), lambda nb: (1, 0, nb, 0)),
            pl.BlockSpec((BND, 1), lambda nb: (nb, 0)),
            pl.BlockSpec((1, F), lambda nb: (0, 0)),
        ],
        out_specs=pl.BlockSpec((BND, T, F), lambda nb: (nb, 0, 0)),
        out_shape=jax.ShapeDtypeStruct((N, T, F), jnp.float32),
    )(acc, acc, dinv, b.reshape(1, F))


def kernel(x, edge_index, W1, b1, W2, b2):
    src = edge_index[0]
    dst = edge_index[1]
    h1 = jnp.pad(jnp.transpose(x, (2, 0, 1)), ((0, 0), (0, NPAD - N), (0, 0)))

    deg = _sc_degree(dst)
    dinv = _tc_dinv(deg)

    hws1 = _tc_mm1(h1, W1, dinv)
    acc1 = _sc_aggregate(hws1.reshape(NC * T * NPAD, FH), src, dst)
    hws2 = _tc_mm2(acc1, W2, dinv, b1)
    acc2 = _sc_aggregate(hws2.reshape(NC * T * NPAD, FH), src, dst)
    return _tc_finalize(acc2, dinv, b2)


# speedup vs baseline: 35.2718x; 1.0428x over previous
import jax
import jax.numpy as jnp
from jax import lax
from jax.experimental import pallas as pl
from jax.experimental.pallas import tpu as pltpu
from jax.experimental.pallas import tpu_sc as plsc

N = 10000
E = 160000
F = 256
FH = 128
T = 4
NPAD = 10240
NC = 2
NS = 16
CH = 128
ECHUNKS = E // CH
KMAX = (ECHUNKS + NS - 1) // NS
RPT = NPAD // NS
RC = 160
EPT = E // NS

BN = 256
NB = NPAD // BN
BND = 400


def _mesh():
    return plsc.VectorSubcoreMesh(
        core_axis_name="c", subcore_axis_name="s", num_cores=NC)


_SC_PARAMS = pltpu.CompilerParams(needs_layout_passes=False)


def _deg_body(dst_hbm, deg_out, dstv, degv, red, outv, shared):
    c = lax.axis_index("c")
    s = lax.axis_index("s")

    @pl.when(c == 0)
    def _build():
        zeros = jnp.zeros((16,), jnp.float32)

        def zbody(i, carry):
            degv[pl.ds(i * 16, 16)] = zeros
            return carry

        lax.fori_loop(0, NPAD // 16, zbody, 0)
        pltpu.sync_copy(dst_hbm.at[pl.ds(s * EPT, EPT)], dstv)
        ones = jnp.ones((16,), jnp.float32)

        def hbody(i, carry):
            idx = dstv[pl.ds(i * 16, 16)]
            plsc.addupdate_scatter(degv, [idx], ones)
            return carry

        lax.fori_loop(0, EPT // 16, hbody, 0)
        pltpu.sync_copy(degv, shared.at[s])

    plsc.subcore_barrier()

    @pl.when(c == 0)
    def _reduce():
        base = s * RPT
        for k in range(NS):
            pltpu.sync_copy(shared.at[k, pl.ds(base, RPT)], red.at[k])

        def rbody(i, carry):
            acc = red[0, pl.ds(i * 16, 16)]
            for k in range(1, NS):
                acc = acc + red[k, pl.ds(i * 16, 16)]
            outv[pl.ds(i * 16, 16)] = acc
            return carry

        lax.fori_loop(0, RPT // 16, rbody, 0)
        pltpu.sync_copy(outv, deg_out.at[pl.ds(base, RPT)])


def _sc_degree(dst):
    return pl.kernel(
        _deg_body,
        out_type=jax.ShapeDtypeStruct((NPAD,), jnp.float32),
        mesh=_mesh(),
        scratch_types=[
            pltpu.VMEM((EPT,), jnp.int32),
            pltpu.VMEM((NPAD,), jnp.float32),
            pltpu.VMEM((NS, RPT), jnp.float32),
            pltpu.VMEM((RPT,), jnp.float32),
            pltpu.VMEM_SHARED((NS, NPAD), jnp.float32),
        ],
        compiler_params=_SC_PARAMS,
    )(dst)


ECPAD = 1280
EPAD = ECPAD * CH
CPT = ECPAD // NS
BODYC = 16
NBODY = CPT // BODYC
HALF = 8


def _agg_body(hws_hbm, srcoff_hbm, dst2d_hbm, out_hbm,
              srcblk, dstblk, rows, shared, gsem, ssem, bsem):
    c = lax.axis_index("c")
    s = lax.axis_index("s")
    rbase = s * RPT
    cs = CPT * s

    def _gather(r, q):
        return pltpu.make_async_copy(
            hws_hbm.at[srcblk.at[q]], rows.at[r], gsem)

    def _scatter(r, q):
        return pltpu.make_async_copy(
            rows.at[r], shared.at[dstblk.at[q]], ssem)

    def _blk(p, row0, half):
        return (
            pltpu.make_async_copy(
                srcoff_hbm.at[p, pl.ds(row0, HALF)],
                srcblk.at[pl.ds(half, HALF)], bsem),
            pltpu.make_async_copy(
                dst2d_hbm.at[pl.ds(row0, HALF)],
                dstblk.at[pl.ds(half, HALF)], bsem),
        )

    def _blk_start(p, row0, half):
        a, b = _blk(p, row0, half)
        a.start()
        b.start()

    def _blk_wait(p, row0, half):
        a, b = _blk(p, row0, half)
        a.wait()
        b.wait()

    for t in range(T):
        p = c * T + t
        tb = p * NPAD
        for j in range(RPT // CH):
            r = j % 2
            pltpu.async_copy(
                hws_hbm.at[pl.ds(tb + rbase + j * CH, CH)],
                rows.at[r], gsem).wait()
            pltpu.sync_copy(rows.at[r], shared.at[pl.ds(rbase + j * CH, CH)])
        _blk_start(p, cs, 0)
        plsc.subcore_barrier()

        def body(b2, carry):
            k0 = cs + b2 * BODYC
            for j in range(BODYC):
                r = j % 2
                if j == 0:
                    _blk_wait(p, k0, 0)
                if j == 2:
                    _blk_start(p, k0 + HALF, HALF)
                if j == 8:
                    _blk_wait(p, k0 + HALF, HALF)
                if j == 10:
                    @pl.when(b2 < NBODY - 1)
                    def _():
                        _blk_start(p, k0 + BODYC, 0)

                if j >= 2:
                    _scatter(r, j - 2).wait()
                else:
                    @pl.when(b2 > 0)
                    def _():
                        _scatter(r, j + BODYC - 2).wait()
                _gather(r, j).start()
                if j >= 1:
                    _gather(1 - r, j - 1).wait()
                    _scatter(1 - r, j - 1).start(add=True)
                else:
                    @pl.when(b2 > 0)
                    def _():
                        _gather(1 - r, BODYC - 1).wait()
                        _scatter(1 - r, BODYC - 1).start(add=True)
            return carry

        lax.fori_loop(0, NBODY, body, 0)
        _gather(1, BODYC - 1).wait()
        _scatter(1, BODYC - 1).start(add=True)
        _scatter(0, BODYC - 2).wait()
        _scatter(1, BODYC - 1).wait()

        plsc.subcore_barrier()

        for j in range(RPT // CH):
            r = j % 2
            pltpu.sync_copy(shared.at[pl.ds(rbase + j * CH, CH)], rows.at[r])
            pltpu.async_copy(
                rows.at[r],
                out_hbm.at[c, t, pl.ds(rbase + j * CH, CH)], gsem).wait()


def _sc_aggregate(hws_flat, srcoff, dst2d):
    return pl.kernel(
        _agg_body,
        out_type=jax.ShapeDtypeStruct((NC, T, NPAD, FH), jnp.float32),
        mesh=_mesh(),
        scratch_types=[
            pltpu.VMEM((BODYC, CH), jnp.int32),
            pltpu.VMEM((BODYC, CH), jnp.int32),
            pltpu.VMEM((2, CH, FH), jnp.float32),
            pltpu.VMEM_SHARED((NPAD, FH), jnp.float32),
            pltpu.SemaphoreType.DMA,
            pltpu.SemaphoreType.DMA,
            pltpu.SemaphoreType.DMA,
        ],
        compiler_params=_SC_PARAMS,
    )(hws_flat, srcoff, dst2d)


def _dinv_body(deg_ref, o_ref):
    ri = lax.broadcasted_iota(jnp.int32, (NPAD // 128, 128), 0)
    ci = lax.broadcasted_iota(jnp.int32, (NPAD // 128, 128), 1)
    node = ri * 128 + ci
    o_ref[...] = jnp.where(node < N, lax.rsqrt(deg_ref[...] + 1.0), 0.0)


def _tc_dinv(deg):
    out = pl.pallas_call(
        _dinv_body,
        out_shape=jax.ShapeDtypeStruct((NPAD // 128, 128), jnp.float32),
    )(deg.reshape(NPAD // 128, 128))
    return out.reshape(NPAD, 1)


def _mm1_body(h_ref, w_ref, dinv_ref, o_ref):
    hw = lax.dot_general(h_ref[0], w_ref[...], (((1,), (1,)), ((), ())),
                         preferred_element_type=jnp.float32)
    hw = hw * dinv_ref[...]
    o_ref[0, 0] = hw[:, :FH]
    o_ref[1, 0] = hw[:, FH:]


def _tc_mm1(h, W, dinv):
    return pl.pallas_call(
        _mm1_body,
        grid=(T, NB),
        in_specs=[
            pl.BlockSpec((1, BN, F), lambda t, nb: (t, nb, 0)),
            pl.BlockSpec((F, F), lambda t, nb: (0, 0)),
            pl.BlockSpec((BN, 1), lambda t, nb: (nb, 0)),
        ],
        out_specs=pl.BlockSpec((NC, 1, BN, FH), lambda t, nb: (0, t, nb, 0)),
        out_shape=jax.ShapeDtypeStruct((NC, T, NPAD, FH), jnp.float32),
    )(h, W, dinv)


def _mm2_body(accl_ref, accr_ref, w_ref, dinv_ref, b_ref, o_ref):
    acc = jnp.concatenate([accl_ref[0, 0], accr_ref[0, 0]], axis=1)
    h = jnp.maximum(acc * dinv_ref[...] + b_ref[...], 0.0)
    hw = lax.dot_general(h, w_ref[...], (((1,), (1,)), ((), ())),
                         preferred_element_type=jnp.float32)
    hw = hw * dinv_ref[...]
    o_ref[0, 0] = hw[:, :FH]
    o_ref[1, 0] = hw[:, FH:]


def _tc_mm2(acc, W, dinv, b):
    return pl.pallas_call(
        _mm2_body,
        grid=(T, NB),
        in_specs=[
            pl.BlockSpec((1, 1, BN, FH), lambda t, nb: (0, t, nb, 0)),
            pl.BlockSpec((1, 1, BN, FH), lambda t, nb: (1, t, nb, 0)),
            pl.BlockSpec((F, F), lambda t, nb: (0, 0)),
            pl.BlockSpec((BN, 1), lambda t, nb: (nb, 0)),
            pl.BlockSpec((1, F), lambda t, nb: (0, 0)),
        ],
        out_specs=pl.BlockSpec((NC, 1, BN, FH), lambda t, nb: (0, t, nb, 0)),
        out_shape=jax.ShapeDtypeStruct((NC, T, NPAD, FH), jnp.float32),
    )(acc, acc, W, dinv, b.reshape(1, F))


def _fin_body(accl_ref, accr_ref, dinv_ref, b_ref, o_ref):
    for t in range(T):
        acc = jnp.concatenate([accl_ref[0, t], accr_ref[0, t]], axis=1)
        o_ref[:, t, :] = jnp.maximum(acc * dinv_ref[...] + b_ref[...], 0.0)


def _tc_finalize(acc, dinv, b):
    return pl.pallas_call(
        _fin_body,
        grid=(N // BND,),
        in_specs=[
            pl.BlockSpec((1, T, BND, FH), lambda nb: (0, 0, nb, 0)),
            pl.BlockSpec((1, T, BND, FH), lambda nb: (1, 0, nb, 0)),
            pl.BlockSpec((BND, 1), lambda nb: (nb, 0)),
            pl.BlockSpec((1, F), lambda nb: (0, 0)),
        ],
        out_specs=pl.BlockSpec((BND, T, F), lambda nb: (nb, 0, 0)),
        out_shape=jax.ShapeDtypeStruct((N, T, F), jnp.float32),
    )(acc, acc, dinv, b.reshape(1, F))


def kernel(x, edge_index, W1, b1, W2, b2):
    src = edge_index[0]
    dst = edge_index[1]
    h1 = jnp.pad(jnp.transpose(x, (2, 0, 1)), ((0, 0), (0, NPAD - N), (0, 0)))
    npad_e = EPAD - E
    pad_src = N + jnp.arange(npad_e, dtype=jnp.int32) % (NPAD - N)
    pad_dst = jnp.arange(npad_e, dtype=jnp.int32) % N
    srcp = jnp.concatenate([src, pad_src])
    dstp = jnp.concatenate([dst, pad_dst])
    planes = jnp.arange(NC * T, dtype=jnp.int32) * NPAD
    srcoff = (srcp[None, :] + planes[:, None]).reshape(NC * T, ECPAD, CH)
    dst2d = dstp.reshape(ECPAD, CH)

    deg = _sc_degree(dst)
    dinv = _tc_dinv(deg)

    hws1 = _tc_mm1(h1, W1, dinv)
    acc1 = _sc_aggregate(hws1.reshape(NC * T * NPAD, FH), srcoff, dst2d)
    hws2 = _tc_mm2(acc1, W2, dinv, b1)
    acc2 = _sc_aggregate(hws2.reshape(NC * T * NPAD, FH), srcoff, dst2d)
    return _tc_finalize(acc2, dinv, b2)


# speedup vs baseline: 37.1279x; 1.0526x over previous
import jax
import jax.numpy as jnp
from jax import lax
from jax.experimental import pallas as pl
from jax.experimental.pallas import tpu as pltpu
from jax.experimental.pallas import tpu_sc as plsc

N = 10000
E = 160000
F = 256
FH = 128
T = 4
NPAD = 10240
NC = 2
NS = 16
CH = 128
ECHUNKS = E // CH
KMAX = (ECHUNKS + NS - 1) // NS
RPT = NPAD // NS
RC = 160
EPT = E // NS

BN = 256
NB = NPAD // BN
BND = 400


def _mesh():
    return plsc.VectorSubcoreMesh(
        core_axis_name="c", subcore_axis_name="s", num_cores=NC)


_SC_PARAMS = pltpu.CompilerParams(needs_layout_passes=False)


def _deg_body(dst_hbm, deg_out, dstv, degv, red, outv, shared):
    c = lax.axis_index("c")
    s = lax.axis_index("s")

    @pl.when(c == 0)
    def _build():
        zeros = jnp.zeros((16,), jnp.float32)

        def zbody(i, carry):
            degv[pl.ds(i * 16, 16)] = zeros
            return carry

        lax.fori_loop(0, NPAD // 16, zbody, 0)
        pltpu.sync_copy(dst_hbm.at[pl.ds(s * EPT, EPT)], dstv)
        ones = jnp.ones((16,), jnp.float32)

        def hbody(i, carry):
            idx = dstv[pl.ds(i * 16, 16)]
            plsc.addupdate_scatter(degv, [idx], ones)
            return carry

        lax.fori_loop(0, EPT // 16, hbody, 0)
        pltpu.sync_copy(degv, shared.at[s])

    plsc.subcore_barrier()

    @pl.when(c == 0)
    def _reduce():
        base = s * RPT
        for k in range(NS):
            pltpu.sync_copy(shared.at[k, pl.ds(base, RPT)], red.at[k])

        def rbody(i, carry):
            acc = red[0, pl.ds(i * 16, 16)]
            for k in range(1, NS):
                acc = acc + red[k, pl.ds(i * 16, 16)]
            outv[pl.ds(i * 16, 16)] = acc
            return carry

        lax.fori_loop(0, RPT // 16, rbody, 0)
        pltpu.sync_copy(outv, deg_out.at[pl.ds(base, RPT)])


def _sc_degree(dst):
    return pl.kernel(
        _deg_body,
        out_type=jax.ShapeDtypeStruct((NPAD,), jnp.float32),
        mesh=_mesh(),
        scratch_types=[
            pltpu.VMEM((EPT,), jnp.int32),
            pltpu.VMEM((NPAD,), jnp.float32),
            pltpu.VMEM((NS, RPT), jnp.float32),
            pltpu.VMEM((RPT,), jnp.float32),
            pltpu.VMEM_SHARED((NS, NPAD), jnp.float32),
        ],
        compiler_params=_SC_PARAMS,
    )(dst)


ECPAD = 1280
EPAD = ECPAD * CH
CPT = ECPAD // NS
BODYC = 16
NBODY = CPT // BODYC
HALF = 8


def _agg_body(hws_hbm, srcoff_hbm, dst2d_hbm, out_hbm,
              srcblk, dstblk, rows, shared, gsem, ssem, bsem):
    c = lax.axis_index("c")
    s = lax.axis_index("s")
    rbase = s * RPT
    cs = CPT * s

    def _gather(r, q):
        return pltpu.make_async_copy(
            hws_hbm.at[srcblk.at[q]], rows.at[r], gsem)

    def _scatter(r, q):
        return pltpu.make_async_copy(
            rows.at[r], shared.at[dstblk.at[q]], ssem)

    def _blk(p, row0, half):
        return (
            pltpu.make_async_copy(
                srcoff_hbm.at[p, pl.ds(row0, HALF)],
                srcblk.at[pl.ds(half, HALF)], bsem),
            pltpu.make_async_copy(
                dst2d_hbm.at[pl.ds(row0, HALF)],
                dstblk.at[pl.ds(half, HALF)], bsem),
        )

    def _blk_start(p, row0, half):
        a, b = _blk(p, row0, half)
        a.start()
        b.start()

    def _blk_wait(p, row0, half):
        a, b = _blk(p, row0, half)
        a.wait()
        b.wait()

    for t in range(T):
        p = c * T + t
        tb = p * NPAD
        pltpu.sync_copy(hws_hbm.at[pl.ds(tb + rbase, RPT)],
                        shared.at[pl.ds(rbase, RPT)])
        _blk_start(p, cs, 0)
        plsc.subcore_barrier()

        def body(b2, carry):
            k0 = cs + b2 * BODYC
            for j in range(BODYC):
                r = j % 2
                if j == 0:
                    _blk_wait(p, k0, 0)
                if j == 2:
                    _blk_start(p, k0 + HALF, HALF)
                if j == 8:
                    _blk_wait(p, k0 + HALF, HALF)
                if j == 10:
                    @pl.when(b2 < NBODY - 1)
                    def _():
                        _blk_start(p, k0 + BODYC, 0)

                if j >= 2:
                    _scatter(r, j - 2).wait()
                else:
                    @pl.when(b2 > 0)
                    def _():
                        _scatter(r, j + BODYC - 2).wait()
                _gather(r, j).start()
                if j >= 1:
                    _gather(1 - r, j - 1).wait()
                    _scatter(1 - r, j - 1).start(add=True)
                else:
                    @pl.when(b2 > 0)
                    def _():
                        _gather(1 - r, BODYC - 1).wait()
                        _scatter(1 - r, BODYC - 1).start(add=True)
            return carry

        lax.fori_loop(0, NBODY, body, 0)
        _gather(1, BODYC - 1).wait()
        _scatter(1, BODYC - 1).start(add=True)
        _scatter(0, BODYC - 2).wait()
        _scatter(1, BODYC - 1).wait()

        plsc.subcore_barrier()

        pltpu.sync_copy(shared.at[pl.ds(rbase, RPT)],
                        out_hbm.at[c, t, pl.ds(rbase, RPT)])


def _sc_aggregate(hws_flat, srcoff, dst2d):
    return pl.kernel(
        _agg_body,
        out_type=jax.ShapeDtypeStruct((NC, T, NPAD, FH), jnp.float32),
        mesh=_mesh(),
        scratch_types=[
            pltpu.VMEM((BODYC, CH), jnp.int32),
            pltpu.VMEM((BODYC, CH), jnp.int32),
            pltpu.VMEM((2, CH, FH), jnp.float32),
            pltpu.VMEM_SHARED((NPAD, FH), jnp.float32),
            pltpu.SemaphoreType.DMA,
            pltpu.SemaphoreType.DMA,
            pltpu.SemaphoreType.DMA,
        ],
        compiler_params=_SC_PARAMS,
    )(hws_flat, srcoff, dst2d)


def _dinv_body(deg_ref, o_ref):
    ri = lax.broadcasted_iota(jnp.int32, (NPAD // 128, 128), 0)
    ci = lax.broadcasted_iota(jnp.int32, (NPAD // 128, 128), 1)
    node = ri * 128 + ci
    o_ref[...] = jnp.where(node < N, lax.rsqrt(deg_ref[...] + 1.0), 0.0)


def _tc_dinv(deg):
    out = pl.pallas_call(
        _dinv_body,
        out_shape=jax.ShapeDtypeStruct((NPAD // 128, 128), jnp.float32),
    )(deg.reshape(NPAD // 128, 128))
    return out.reshape(NPAD, 1)


def _mm1_body(h_ref, w_ref, dinv_ref, o_ref):
    hw = lax.dot_general(h_ref[0], w_ref[...], (((1,), (1,)), ((), ())),
                         preferred_element_type=jnp.float32)
    hw = hw * dinv_ref[...]
    o_ref[0, 0] = hw[:, :FH]
    o_ref[1, 0] = hw[:, FH:]


def _tc_mm1(h, W, dinv):
    return pl.pallas_call(
        _mm1_body,
        grid=(T, NB),
        in_specs=[
            pl.BlockSpec((1, BN, F), lambda t, nb: (t, nb, 0)),
            pl.BlockSpec((F, F), lambda t, nb: (0, 0)),
            pl.BlockSpec((BN, 1), lambda t, nb: (nb, 0)),
        ],
        out_specs=pl.BlockSpec((NC, 1, BN, FH), lambda t, nb: (0, t, nb, 0)),
        out_shape=jax.ShapeDtypeStruct((NC, T, NPAD, FH), jnp.float32),
    )(h, W, dinv)


def _mm2_body(accl_ref, accr_ref, w_ref, dinv_ref, b_ref, o_ref):
    acc = jnp.concatenate([accl_ref[0, 0], accr_ref[0, 0]], axis=1)
    h = jnp.maximum(acc * dinv_ref[...] + b_ref[...], 0.0)
    hw = lax.dot_general(h, w_ref[...], (((1,), (1,)), ((), ())),
                         preferred_element_type=jnp.float32)
    hw = hw * dinv_ref[...]
    o_ref[0, 0] = hw[:, :FH]
    o_ref[1, 0] = hw[:, FH:]


def _tc_mm2(acc, W, dinv, b):
    return pl.pallas_call(
        _mm2_body,
        grid=(T, NB),
        in_specs=[
            pl.BlockSpec((1, 1, BN, FH), lambda t, nb: (0, t, nb, 0)),
            pl.BlockSpec((1, 1, BN, FH), lambda t, nb: (1, t, nb, 0)),
            pl.BlockSpec((F, F), lambda t, nb: (0, 0)),
            pl.BlockSpec((BN, 1), lambda t, nb: (nb, 0)),
            pl.BlockSpec((1, F), lambda t, nb: (0, 0)),
        ],
        out_specs=pl.BlockSpec((NC, 1, BN, FH), lambda t, nb: (0, t, nb, 0)),
        out_shape=jax.ShapeDtypeStruct((NC, T, NPAD, FH), jnp.float32),
    )(acc, acc, W, dinv, b.reshape(1, F))


def _fin_body(accl_ref, accr_ref, dinv_ref, b_ref, o_ref):
    for t in range(T):
        acc = jnp.concatenate([accl_ref[0, t], accr_ref[0, t]], axis=1)
        o_ref[:, t, :] = jnp.maximum(acc * dinv_ref[...] + b_ref[...], 0.0)


def _tc_finalize(acc, dinv, b):
    return pl.pallas_call(
        _fin_body,
        grid=(N // BND,),
        in_specs=[
            pl.BlockSpec((1, T, BND, FH), lambda nb: (0, 0, nb, 0)),
            pl.BlockSpec((1, T, BND, FH), lambda nb: (1, 0, nb, 0)),
            pl.BlockSpec((BND, 1), lambda nb: (nb, 0)),
            pl.BlockSpec((1, F), lambda nb: (0, 0)),
        ],
        out_specs=pl.BlockSpec((BND, T, F), lambda nb: (nb, 0, 0)),
        out_shape=jax.ShapeDtypeStruct((N, T, F), jnp.float32),
    )(acc, acc, dinv, b.reshape(1, F))


def kernel(x, edge_index, W1, b1, W2, b2):
    src = edge_index[0]
    dst = edge_index[1]
    h1 = jnp.pad(jnp.transpose(x, (2, 0, 1)), ((0, 0), (0, NPAD - N), (0, 0)))
    npad_e = EPAD - E
    pad_src = N + jnp.arange(npad_e, dtype=jnp.int32) % (NPAD - N)
    pad_dst = jnp.arange(npad_e, dtype=jnp.int32) % N
    srcp = jnp.concatenate([src, pad_src])
    dstp = jnp.concatenate([dst, pad_dst])
    planes = jnp.arange(NC * T, dtype=jnp.int32) * NPAD
    srcoff = (srcp[None, :] + planes[:, None]).reshape(NC * T, ECPAD, CH)
    dst2d = dstp.reshape(ECPAD, CH)

    deg = _sc_degree(dst)
    dinv = _tc_dinv(deg)

    hws1 = _tc_mm1(h1, W1, dinv)
    acc1 = _sc_aggregate(hws1.reshape(NC * T * NPAD, FH), srcoff, dst2d)
    hws2 = _tc_mm2(acc1, W2, dinv, b1)
    acc2 = _sc_aggregate(hws2.reshape(NC * T * NPAD, FH), srcoff, dst2d)
    return _tc_finalize(acc2, dinv, b2)
